# baseline (speedup 1.0000x reference)
import functools
import math

import jax
import jax.numpy as jnp
import numpy as np
from jax import lax
from jax.experimental import pallas as pl
from jax.experimental.pallas import tpu as pltpu
from jax.experimental.pallas import tpu_sc as plsc

_B, _H, _W = 4, 224, 224
_R = 8
_NUM = 200
_HB, _WB = _H // _R, _W // _R
_NBLK = _HB * _WB
_NV = _NBLK // 16
_N_DESC, _D_DESC = 256, 128
_NPIX = float(_B * _H * _W)
_NCOS = float(_B * _N_DESC * _N_DESC)


def _gauss_taps(ksize=7, sigma=1.0):
    x = np.arange(ksize, dtype=np.float64) - (ksize - 1) / 2.0
    g = np.exp(-(x ** 2) / (2.0 * sigma ** 2))
    g = g / g.sum()
    return [float(v) for v in g]


def _rpad(x, axis, p):
    n = x.shape[axis]
    parts = [lax.slice_in_dim(x, p - k, p - k + 1, axis=axis)
             for k in range(p)]
    parts.append(x)
    parts += [lax.slice_in_dim(x, n - 2 - k, n - 1 - k, axis=axis)
              for k in range(p)]
    return jnp.concatenate(parts, axis=axis)


def _conv1(x, taps, axis):
    p = len(taps) // 2
    n = x.shape[axis]
    xp = _rpad(x, axis, p)
    acc = None
    for k, w in enumerate(taps):
        if w == 0.0:
            continue
        s = lax.slice_in_dim(xp, k, k + n, axis=axis)
        term = s if w == 1.0 else s * w
        acc = term if acc is None else acc + term
    return acc


def _maxpool1(x, axis, ks=5):
    p = ks // 2
    n = x.shape[axis]
    shp = list(x.shape)
    shp[axis] = p
    pad = jnp.full(shp, -jnp.inf, x.dtype)
    xp = jnp.concatenate([pad, x, pad], axis=axis)
    acc = None
    for k in range(ks):
        s = lax.slice_in_dim(xp, k, k + n, axis=axis)
        acc = s if acc is None else jnp.maximum(acc, s)
    return acc


def _dense_kernel(imgs_ref, sd_ref, desc_ref, part_ref, cbits_ref, sb_ref):
    imgs = imgs_ref[...]
    gray = (0.299 * imgs[:, 0] + 0.587 * imgs[:, 1]
            + 0.114 * imgs[:, 2])

    dx = _conv1(_conv1(gray, [-1.0, 0.0, 1.0], axis=2),
                [1.0, 2.0, 1.0], axis=1) * 0.125
    dy = _conv1(_conv1(gray, [-1.0, 0.0, 1.0], axis=1),
                [1.0, 2.0, 1.0], axis=2) * 0.125

    g7 = _gauss_taps()

    def blur(z):
        return _conv1(_conv1(z, g7, axis=2), g7, axis=1)

    dx2 = blur(dx * dx)
    dy2 = blur(dy * dy)
    dxy = blur(dx * dy)
    det = dx2 * dy2 - dxy * dxy
    trace = dx2 + dy2
    e = 0.5 * (trace - jnp.sqrt(jnp.maximum(trace * trace - 4.0 * det, 0.0)
                                + 1e-12))

    mp = _maxpool1(_maxpool1(e, axis=1), axis=2)
    nms = e * (e == mp).astype(e.dtype)

    xh = jnp.max(nms.reshape(_B, _HB, _R, _W), axis=2)
    xt = jnp.swapaxes(xh, 1, 2)
    c_t = jnp.max(xt.reshape(_B, _WB, _R, _HB), axis=2)
    bm_t = jnp.broadcast_to(c_t[:, :, None, :],
                            (_B, _WB, _R, _HB)).reshape(_B, _W, _HB)
    bm_h = jnp.swapaxes(bm_t, 1, 2)
    bmax = jnp.broadcast_to(bm_h[:, :, None, :],
                            (_B, _HB, _R, _W)).reshape(_B, _H, _W)

    cand = jnp.maximum(c_t, 0.0).reshape(_B, _NBLK)
    cbits_ref[...] = lax.bitcast_convert_type(cand, jnp.int32)

    s = sd_ref[...].reshape(_B, _H, _W)
    surv = (nms > 0.0) & (nms == bmax)
    ms = jnp.where(surv, s, 0.0)
    mh = jnp.sum(ms.reshape(_B, _HB, _R, _W), axis=2)
    mt = jnp.swapaxes(mh, 1, 2)
    sb_t = jnp.sum(mt.reshape(_B, _WB, _R, _HB), axis=2)
    sb_ref[...] = sb_t.reshape(_B, _NBLK)

    a_sum = jnp.sum(jnp.maximum(s, 0.0)
                    + jnp.log(1.0 + jnp.exp(-jnp.abs(s))))

    d = desc_ref[...]
    cos_sum = jnp.float32(0.0)
    for b in range(_B):
        db = d[b]
        sq = jnp.sum(db * db, axis=1, keepdims=True)
        nr = jnp.sqrt(sq)
        denom = jnp.maximum(nr * jnp.transpose(nr), 1e-8)
        dots = lax.dot_general(db, db, (((1,), (1,)), ((), ())),
                               preferred_element_type=jnp.float32)
        cos_sum = cos_sum + jnp.sum(jnp.maximum(dots, 0.0) / denom)

    part = a_sum / _NPIX + cos_sum / _NCOS
    part_ref[...] = part.reshape(1, 1)


def _select_sc_kernel(cbits_hbm, sb_hbm, out_hbm, cb_v, sb_v, res_v):
    wid = lax.axis_index("s") * 2 + lax.axis_index("c")

    @pl.when(wid < _B)
    def _():
        b = wid
        pltpu.sync_copy(cbits_hbm.at[b], cb_v)
        pltpu.sync_copy(sb_hbm.at[b], sb_v)

        def count_lt(midv):
            def body(i, cnt):
                v = cb_v[pl.ds(i * 16, 16)]
                return cnt + lax.shift_right_logical(v - midv, 31)
            return lax.fori_loop(0, _NV, body, jnp.zeros((16,), jnp.int32))

        lane_iota = lax.iota(jnp.int32, 16)
        perms = [jnp.bitwise_xor(lane_iota, sh).reshape(16, 1)
                 for sh in (8, 4, 2, 1)]
        gdn = lax.GatherDimensionNumbers(offset_dims=(),
                                         collapsed_slice_dims=(0,),
                                         start_index_map=(0,))

        def lane_total(x):
            for perm in perms:
                x = x + lax.gather(x, perm, gdn, (1,),
                                   mode=lax.GatherScatterMode.PROMISE_IN_BOUNDS)
            return x

        lo = jnp.full((16,), 1, jnp.int32)
        hi = jnp.full((16,), 0x7F7FFFFF, jnp.int32)
        one = jnp.full((16,), 1, jnp.int32)
        n_ge_target = jnp.full((16,), _NBLK - _NUM, jnp.int32)
        for _step in range(31):
            mid = lo + lax.shift_right_logical(hi - lo + one, 1)
            n_lt = lane_total(count_lt(mid))
            ok = n_lt <= n_ge_target
            lo = jnp.where(ok, mid, lo)
            hi = jnp.where(ok, hi, mid - one)
        lov = lo

        acc = jnp.zeros((16,), jnp.float32)
        for i in range(_NV):
            v = cb_v[pl.ds(i * 16, 16)]
            sbv = sb_v[pl.ds(i * 16, 16)]
            acc = acc + jnp.where(v >= lov, sbv, 0.0)
        res_v[...] = acc
        pltpu.sync_copy(res_v, out_hbm.at[b])


def kernel(descriptors, scores, scores_dense, imgs):
    del scores
    part, cbits, sb = pl.pallas_call(
        _dense_kernel,
        out_shape=(
            jax.ShapeDtypeStruct((1, 1), jnp.float32),
            jax.ShapeDtypeStruct((_B, _NBLK), jnp.int32),
            jax.ShapeDtypeStruct((_B, _NBLK), jnp.float32),
        ),
    )(imgs, scores_dense, descriptors)

    sel = pl.kernel(
        _select_sc_kernel,
        out_type=jax.ShapeDtypeStruct((_B, 16), jnp.float32),
        scratch_types=[
            pltpu.VMEM((_NBLK,), jnp.int32),
            pltpu.VMEM((_NBLK,), jnp.float32),
            pltpu.VMEM((16,), jnp.float32),
        ],
        mesh=plsc.VectorSubcoreMesh(core_axis_name="c", subcore_axis_name="s"),
    )(cbits, sb)

    return part[0, 0] - jnp.sum(sel) / _NPIX


# speedup vs baseline: 1.0930x; 1.0930x over previous
import functools
import math

import jax
import jax.numpy as jnp
import numpy as np
from jax import lax
from jax.experimental import pallas as pl
from jax.experimental.pallas import tpu as pltpu
from jax.experimental.pallas import tpu_sc as plsc

_B, _H, _W = 4, 224, 224
_R = 8
_NUM = 200
_HB, _WB = _H // _R, _W // _R
_NBLK = _HB * _WB
_NV = _NBLK // 16
_N_DESC, _D_DESC = 256, 128
_NPIX = float(_B * _H * _W)
_NCOS = float(_B * _N_DESC * _N_DESC)


def _gauss_taps(ksize=7, sigma=1.0):
    x = np.arange(ksize, dtype=np.float64) - (ksize - 1) / 2.0
    g = np.exp(-(x ** 2) / (2.0 * sigma ** 2))
    g = g / g.sum()
    return [float(v) for v in g]


def _rpad(x, axis, p):
    n = x.shape[axis]
    parts = [lax.slice_in_dim(x, p - k, p - k + 1, axis=axis)
             for k in range(p)]
    parts.append(x)
    parts += [lax.slice_in_dim(x, n - 2 - k, n - 1 - k, axis=axis)
              for k in range(p)]
    return jnp.concatenate(parts, axis=axis)


def _conv1(x, taps, axis):
    p = len(taps) // 2
    n = x.shape[axis]
    xp = _rpad(x, axis, p)
    acc = None
    for k, w in enumerate(taps):
        if w == 0.0:
            continue
        s = lax.slice_in_dim(xp, k, k + n, axis=axis)
        term = s if w == 1.0 else s * w
        acc = term if acc is None else acc + term
    return acc


def _maxpool1(x, axis, ks=5):
    p = ks // 2
    n = x.shape[axis]
    shp = list(x.shape)
    shp[axis] = p
    pad = jnp.full(shp, -jnp.inf, x.dtype)
    xp = jnp.concatenate([pad, x, pad], axis=axis)
    acc = None
    for k in range(ks):
        s = lax.slice_in_dim(xp, k, k + n, axis=axis)
        acc = s if acc is None else jnp.maximum(acc, s)
    return acc


def _dense_kernel(imgs_ref, sd_ref, desc_ref, part_ref, cbits_ref, sb_ref):
    imgs = imgs_ref[...]
    gray = (0.299 * imgs[:, 0] + 0.587 * imgs[:, 1]
            + 0.114 * imgs[:, 2])

    dx = _conv1(_conv1(gray, [-1.0, 0.0, 1.0], axis=2),
                [1.0, 2.0, 1.0], axis=1) * 0.125
    dy = _conv1(_conv1(gray, [-1.0, 0.0, 1.0], axis=1),
                [1.0, 2.0, 1.0], axis=2) * 0.125

    g7 = _gauss_taps()

    def blur(z):
        return _conv1(_conv1(z, g7, axis=2), g7, axis=1)

    dx2 = blur(dx * dx)
    dy2 = blur(dy * dy)
    dxy = blur(dx * dy)
    det = dx2 * dy2 - dxy * dxy
    trace = dx2 + dy2
    e = 0.5 * (trace - jnp.sqrt(jnp.maximum(trace * trace - 4.0 * det, 0.0)
                                + 1e-12))

    mp = _maxpool1(_maxpool1(e, axis=1), axis=2)
    nms = e * (e == mp).astype(e.dtype)

    xh = jnp.max(nms.reshape(_B, _HB, _R, _W), axis=2)
    xt = jnp.swapaxes(xh, 1, 2)
    c_t = jnp.max(xt.reshape(_B, _WB, _R, _HB), axis=2)
    bm_t = jnp.broadcast_to(c_t[:, :, None, :],
                            (_B, _WB, _R, _HB)).reshape(_B, _W, _HB)
    bm_h = jnp.swapaxes(bm_t, 1, 2)
    bmax = jnp.broadcast_to(bm_h[:, :, None, :],
                            (_B, _HB, _R, _W)).reshape(_B, _H, _W)

    cand = jnp.maximum(c_t, 0.0).reshape(_B, _NBLK)
    cbits_ref[...] = lax.bitcast_convert_type(cand, jnp.int32)

    s = sd_ref[...].reshape(_B, _H, _W)
    surv = (nms > 0.0) & (nms == bmax)
    ms = jnp.where(surv, s, 0.0)
    mh = jnp.sum(ms.reshape(_B, _HB, _R, _W), axis=2)
    mt = jnp.swapaxes(mh, 1, 2)
    sb_t = jnp.sum(mt.reshape(_B, _WB, _R, _HB), axis=2)
    sb_ref[...] = sb_t.reshape(_B, _NBLK)

    a_sum = jnp.sum(jnp.maximum(s, 0.0)
                    + jnp.log(1.0 + jnp.exp(-jnp.abs(s))))

    d = desc_ref[...]
    cos_sum = jnp.float32(0.0)
    for b in range(_B):
        db = d[b]
        sq = jnp.sum(db * db, axis=1, keepdims=True)
        nr = jnp.sqrt(sq)
        denom = jnp.maximum(nr * jnp.transpose(nr), 1e-8)
        dots = lax.dot_general(db, db, (((1,), (1,)), ((), ())),
                               preferred_element_type=jnp.float32)
        cos_sum = cos_sum + jnp.sum(jnp.maximum(dots, 0.0) / denom)

    part = a_sum / _NPIX + cos_sum / _NCOS
    part_ref[...] = part.reshape(1, 1)


def _select_sc_kernel(cbits_hbm, sb_hbm, out_hbm, cb_v, sb_v, res_v):
    wid = lax.axis_index("s") * 2 + lax.axis_index("c")

    @pl.when(wid < _B)
    def _():
        b = wid
        pltpu.sync_copy(cbits_hbm.at[b], cb_v)
        pltpu.sync_copy(sb_hbm.at[b], sb_v)

        def count_lt(midv):
            def body(i, cnt):
                base = i * 112
                for j in range(7):
                    v = cb_v[pl.ds(base + j * 16, 16)]
                    cnt = cnt + lax.shift_right_logical(v - midv, 31)
                return cnt
            return lax.fori_loop(0, 7, body, jnp.zeros((16,), jnp.int32))

        lane_iota = lax.iota(jnp.int32, 16)
        perms = [jnp.bitwise_xor(lane_iota, sh).reshape(16, 1)
                 for sh in (8, 4, 2, 1)]
        gdn = lax.GatherDimensionNumbers(offset_dims=(),
                                         collapsed_slice_dims=(0,),
                                         start_index_map=(0,))

        def lane_total(x):
            for perm in perms:
                x = x + lax.gather(x, perm, gdn, (1,),
                                   mode=lax.GatherScatterMode.PROMISE_IN_BOUNDS)
            return x

        lo = jnp.full((16,), 1, jnp.int32)
        hi = jnp.full((16,), 0x7F7FFFFF, jnp.int32)
        one = jnp.full((16,), 1, jnp.int32)
        n_ge_target = jnp.full((16,), _NBLK - _NUM, jnp.int32)
        for _step in range(31):
            mid = lo + lax.shift_right_logical(hi - lo + one, 1)
            n_lt = lane_total(count_lt(mid))
            ok = n_lt <= n_ge_target
            lo = jnp.where(ok, mid, lo)
            hi = jnp.where(ok, hi, mid - one)
        lov = lo

        acc = jnp.zeros((16,), jnp.float32)
        for i in range(_NV):
            v = cb_v[pl.ds(i * 16, 16)]
            sbv = sb_v[pl.ds(i * 16, 16)]
            acc = acc + jnp.where(v >= lov, sbv, 0.0)
        res_v[...] = acc
        pltpu.sync_copy(res_v, out_hbm.at[b])


def kernel(descriptors, scores, scores_dense, imgs):
    del scores
    part, cbits, sb = pl.pallas_call(
        _dense_kernel,
        out_shape=(
            jax.ShapeDtypeStruct((1, 1), jnp.float32),
            jax.ShapeDtypeStruct((_B, _NBLK), jnp.int32),
            jax.ShapeDtypeStruct((_B, _NBLK), jnp.float32),
        ),
    )(imgs, scores_dense, descriptors)

    sel = pl.kernel(
        _select_sc_kernel,
        out_type=jax.ShapeDtypeStruct((_B, 16), jnp.float32),
        scratch_types=[
            pltpu.VMEM((_NBLK,), jnp.int32),
            pltpu.VMEM((_NBLK,), jnp.float32),
            pltpu.VMEM((16,), jnp.float32),
        ],
        mesh=plsc.VectorSubcoreMesh(core_axis_name="c", subcore_axis_name="s"),
    )(cbits, sb)

    return part[0, 0] - jnp.sum(sel) / _NPIX


# speedup vs baseline: 1.1275x; 1.0315x over previous
import functools
import math

import jax
import jax.numpy as jnp
import numpy as np
from jax import lax
from jax.experimental import pallas as pl
from jax.experimental.pallas import tpu as pltpu
from jax.experimental.pallas import tpu_sc as plsc

_B, _H, _W = 4, 224, 224
_R = 8
_NUM = 200
_HB, _WB = _H // _R, _W // _R
_NBLK = _HB * _WB
_NV = _NBLK // 16
_N_DESC, _D_DESC = 256, 128
_NPIX = float(_B * _H * _W)
_NCOS = float(_B * _N_DESC * _N_DESC)


def _gauss_taps(ksize=7, sigma=1.0):
    x = np.arange(ksize, dtype=np.float64) - (ksize - 1) / 2.0
    g = np.exp(-(x ** 2) / (2.0 * sigma ** 2))
    g = g / g.sum()
    return [float(v) for v in g]


def _rpad(x, axis, p):
    n = x.shape[axis]
    parts = [lax.slice_in_dim(x, p - k, p - k + 1, axis=axis)
             for k in range(p)]
    parts.append(x)
    parts += [lax.slice_in_dim(x, n - 2 - k, n - 1 - k, axis=axis)
              for k in range(p)]
    return jnp.concatenate(parts, axis=axis)


def _conv1(x, taps, axis):
    p = len(taps) // 2
    n = x.shape[axis]
    xp = _rpad(x, axis, p)
    acc = None
    for k, w in enumerate(taps):
        if w == 0.0:
            continue
        s = lax.slice_in_dim(xp, k, k + n, axis=axis)
        term = s if w == 1.0 else s * w
        acc = term if acc is None else acc + term
    return acc


def _maxpool1(x, axis, ks=5):
    p = ks // 2
    n = x.shape[axis]
    shp = list(x.shape)
    shp[axis] = p
    pad = jnp.full(shp, -jnp.inf, x.dtype)
    xp = jnp.concatenate([pad, x, pad], axis=axis)
    acc = None
    for k in range(ks):
        s = lax.slice_in_dim(xp, k, k + n, axis=axis)
        acc = s if acc is None else jnp.maximum(acc, s)
    return acc


def _corners_kernel(imgs_ref, sd_ref, cbits_ref, sb_ref):
    imgs = imgs_ref[...]
    gray = (0.299 * imgs[:, 0] + 0.587 * imgs[:, 1]
            + 0.114 * imgs[:, 2])

    dx = _conv1(_conv1(gray, [-1.0, 0.0, 1.0], axis=2),
                [1.0, 2.0, 1.0], axis=1) * 0.125
    dy = _conv1(_conv1(gray, [-1.0, 0.0, 1.0], axis=1),
                [1.0, 2.0, 1.0], axis=2) * 0.125

    g7 = _gauss_taps()

    def blur(z):
        return _conv1(_conv1(z, g7, axis=2), g7, axis=1)

    dx2 = blur(dx * dx)
    dy2 = blur(dy * dy)
    dxy = blur(dx * dy)
    det = dx2 * dy2 - dxy * dxy
    trace = dx2 + dy2
    e = 0.5 * (trace - jnp.sqrt(jnp.maximum(trace * trace - 4.0 * det, 0.0)
                                + 1e-12))

    mp = _maxpool1(_maxpool1(e, axis=1), axis=2)
    nms = e * (e == mp).astype(e.dtype)

    xh = jnp.max(nms.reshape(_B, _HB, _R, _W), axis=2)
    xt = jnp.swapaxes(xh, 1, 2)
    c_t = jnp.max(xt.reshape(_B, _WB, _R, _HB), axis=2)
    bm_t = jnp.broadcast_to(c_t[:, :, None, :],
                            (_B, _WB, _R, _HB)).reshape(_B, _W, _HB)
    bm_h = jnp.swapaxes(bm_t, 1, 2)
    bmax = jnp.broadcast_to(bm_h[:, :, None, :],
                            (_B, _HB, _R, _W)).reshape(_B, _H, _W)

    cand = jnp.maximum(c_t, 0.0).reshape(_B, _NBLK)
    cbits_ref[...] = lax.bitcast_convert_type(cand, jnp.int32)

    s = sd_ref[...].reshape(_B, _H, _W)
    surv = (nms > 0.0) & (nms == bmax)
    ms = jnp.where(surv, s, 0.0)
    mh = jnp.sum(ms.reshape(_B, _HB, _R, _W), axis=2)
    mt = jnp.swapaxes(mh, 1, 2)
    sb_t = jnp.sum(mt.reshape(_B, _WB, _R, _HB), axis=2)
    sb_ref[...] = sb_t.reshape(_B, _NBLK)


def _partial_kernel(sd_ref, desc_ref, part_ref):
    s = sd_ref[...].reshape(_B, _H, _W)
    a_sum = jnp.sum(jnp.maximum(s, 0.0)
                    + jnp.log(1.0 + jnp.exp(-jnp.abs(s))))

    d = desc_ref[...]
    cos_sum = jnp.float32(0.0)
    for b in range(_B):
        db = d[b]
        sq = jnp.sum(db * db, axis=1, keepdims=True)
        nr = jnp.sqrt(sq)
        denom = jnp.maximum(nr * jnp.transpose(nr), 1e-8)
        dots = lax.dot_general(db, db, (((1,), (1,)), ((), ())),
                               preferred_element_type=jnp.float32)
        cos_sum = cos_sum + jnp.sum(jnp.maximum(dots, 0.0) / denom)

    part = a_sum / _NPIX + cos_sum / _NCOS
    part_ref[...] = part.reshape(1, 1)


def _select_sc_kernel(cbits_hbm, sb_hbm, out_hbm, cb_v, sb_v, res_v):
    wid = lax.axis_index("s") * 2 + lax.axis_index("c")

    @pl.when(wid < _B)
    def _():
        b = wid
        pltpu.sync_copy(cbits_hbm.at[b], cb_v)
        pltpu.sync_copy(sb_hbm.at[b], sb_v)

        def count_lt(midv):
            def body(i, cnt):
                base = i * 112
                for j in range(7):
                    v = cb_v[pl.ds(base + j * 16, 16)]
                    cnt = cnt + lax.shift_right_logical(v - midv, 31)
                return cnt
            return lax.fori_loop(0, 7, body, jnp.zeros((16,), jnp.int32))

        lane_iota = lax.iota(jnp.int32, 16)
        perms = [jnp.bitwise_xor(lane_iota, sh).reshape(16, 1)
                 for sh in (8, 4, 2, 1)]
        gdn = lax.GatherDimensionNumbers(offset_dims=(),
                                         collapsed_slice_dims=(0,),
                                         start_index_map=(0,))

        def lane_total(x):
            for perm in perms:
                x = x + lax.gather(x, perm, gdn, (1,),
                                   mode=lax.GatherScatterMode.PROMISE_IN_BOUNDS)
            return x

        lo = jnp.full((16,), 1, jnp.int32)
        hi = jnp.full((16,), 0x7F7FFFFF, jnp.int32)
        one = jnp.full((16,), 1, jnp.int32)
        n_ge_target = jnp.full((16,), _NBLK - _NUM, jnp.int32)
        for _step in range(31):
            mid = lo + lax.shift_right_logical(hi - lo + one, 1)
            n_lt = lane_total(count_lt(mid))
            ok = n_lt <= n_ge_target
            lo = jnp.where(ok, mid, lo)
            hi = jnp.where(ok, hi, mid - one)
        lov = lo

        acc = jnp.zeros((16,), jnp.float32)
        for i in range(_NV):
            v = cb_v[pl.ds(i * 16, 16)]
            sbv = sb_v[pl.ds(i * 16, 16)]
            acc = acc + jnp.where(v >= lov, sbv, 0.0)
        res_v[...] = acc
        pltpu.sync_copy(res_v, out_hbm.at[b])


def kernel(descriptors, scores, scores_dense, imgs):
    del scores
    cbits, sb = pl.pallas_call(
        _corners_kernel,
        out_shape=(
            jax.ShapeDtypeStruct((_B, _NBLK), jnp.int32),
            jax.ShapeDtypeStruct((_B, _NBLK), jnp.float32),
        ),
    )(imgs, scores_dense)

    sel = pl.kernel(
        _select_sc_kernel,
        out_type=jax.ShapeDtypeStruct((_B, 16), jnp.float32),
        scratch_types=[
            pltpu.VMEM((_NBLK,), jnp.int32),
            pltpu.VMEM((_NBLK,), jnp.float32),
            pltpu.VMEM((16,), jnp.float32),
        ],
        mesh=plsc.VectorSubcoreMesh(core_axis_name="c", subcore_axis_name="s"),
    )(cbits, sb)

    part = pl.pallas_call(
        _partial_kernel,
        out_shape=jax.ShapeDtypeStruct((1, 1), jnp.float32),
    )(scores_dense, descriptors)

    return part[0, 0] - jnp.sum(sel) / _NPIX
